# baseline (speedup 1.0000x reference)
import functools

import jax
import jax.numpy as jnp
from jax import lax
from jax.experimental import pallas as pl
from jax.experimental.pallas import tpu as pltpu
from jax.experimental.pallas import tpu_sc as plsc

BATCH = 16384
EMBED = 32
REGS = 0.01

NC = 2
NS = 16
LANES = 16
NW = NC * NS
BPW = BATCH // NW
GROUPS = BPW // LANES

_SC_MESH = plsc.VectorSubcoreMesh(core_axis_name="c", subcore_axis_name="s")


@functools.partial(
    pl.kernel,
    out_type=[
        jax.ShapeDtypeStruct((BATCH,), jnp.float32),
        jax.ShapeDtypeStruct((BATCH,), jnp.float32),
        jax.ShapeDtypeStruct((NW, LANES), jnp.float32),
    ],
    mesh=_SC_MESH,
    compiler_params=pltpu.CompilerParams(use_tc_tiling_on_sc=False,
                                         needs_layout_passes=False),
    scratch_types=[
        pltpu.VMEM((BPW,), jnp.int32),
        pltpu.VMEM((BPW,), jnp.int32),
        pltpu.VMEM((BPW,), jnp.int32),
        pltpu.VMEM((BPW, EMBED), jnp.float32),
        pltpu.VMEM((BPW, EMBED), jnp.float32),
        pltpu.VMEM((BPW, EMBED), jnp.float32),
        pltpu.VMEM((BPW,), jnp.float32),
        pltpu.VMEM((BPW,), jnp.float32),
        pltpu.VMEM((BPW,), jnp.float32),
        pltpu.VMEM((BPW,), jnp.float32),
        pltpu.VMEM((LANES,), jnp.float32),
        pltpu.SemaphoreType.DMA,
    ],
)
def _sc_lookup(user_hbm, pos_hbm, neg_hbm, uemb_hbm, iemb_hbm, bias_hbm,
               plog_hbm, nlog_hbm, reg_hbm,
               uidx, pidx, nidx, urows, prows, nrows, pbias, nbias,
               plog_v, nlog_v, reg_v, sem):
  wid = lax.axis_index("s") * NC + lax.axis_index("c")
  base = wid * BPW

  pltpu.sync_copy(user_hbm.at[pl.ds(base, BPW)], uidx)
  pltpu.sync_copy(pos_hbm.at[pl.ds(base, BPW)], pidx)
  pltpu.sync_copy(neg_hbm.at[pl.ds(base, BPW)], nidx)

  copies = [
      pltpu.async_copy(uemb_hbm.at[uidx], urows, sem),
      pltpu.async_copy(iemb_hbm.at[pidx], prows, sem),
      pltpu.async_copy(iemb_hbm.at[nidx], nrows, sem),
      pltpu.async_copy(bias_hbm.at[pidx], pbias, sem),
      pltpu.async_copy(bias_hbm.at[nidx], nbias, sem),
  ]
  for c in copies:
    c.wait()

  iota = lax.iota(jnp.int32, LANES)

  def group_body(g, acc_reg):
    rows = g * LANES + iota
    accp = jnp.zeros((LANES,), jnp.float32)
    accn = jnp.zeros((LANES,), jnp.float32)
    accr = acc_reg
    for d in range(EMBED):
      dv = jnp.full((LANES,), d, jnp.int32)
      ud = plsc.load_gather(urows, [rows, dv])
      pd = plsc.load_gather(prows, [rows, dv])
      nd = plsc.load_gather(nrows, [rows, dv])
      accp = accp + ud * pd
      accn = accn + ud * nd
      accr = accr + (ud * ud + ud * ud + pd * pd + nd * nd)
    off = g * LANES
    plog_v[pl.ds(off, LANES)] = accp + pbias[pl.ds(off, LANES)]
    nlog_v[pl.ds(off, LANES)] = accn + nbias[pl.ds(off, LANES)]
    return accr

  acc_reg = lax.fori_loop(0, GROUPS, group_body,
                          jnp.zeros((LANES,), jnp.float32))
  reg_v[...] = acc_reg

  pltpu.sync_copy(plog_v, plog_hbm.at[pl.ds(base, BPW)])
  pltpu.sync_copy(nlog_v, nlog_hbm.at[pl.ds(base, BPW)])
  pltpu.sync_copy(reg_v, reg_hbm.at[wid])


def _loss_body(plog_ref, nlog_ref, reg_ref, cls_ref, reg_out_ref):
  pos_l = plog_ref[...]
  neg_l = nlog_ref[...]
  pos_bce = (jnp.maximum(pos_l, 0.0) - pos_l
             + jnp.log1p(jnp.exp(-jnp.abs(pos_l))))
  neg_bce = jnp.maximum(neg_l, 0.0) + jnp.log1p(jnp.exp(-jnp.abs(neg_l)))
  cls_ref[...] = (jnp.mean(pos_bce) + jnp.mean(neg_bce)).reshape(1, 1)
  reg_out_ref[...] = ((REGS * 0.5) * jnp.sum(reg_ref[...])).reshape(1, 1)


def kernel(user, pos, neg, user_embedding, item_embedding, bias):
  user = user.astype(jnp.int32)
  pos = pos.astype(jnp.int32)
  neg = neg.astype(jnp.int32)
  plog, nlog, regs = _sc_lookup(user, pos, neg,
                                user_embedding, item_embedding, bias)
  cls, reg = pl.pallas_call(
      _loss_body,
      out_shape=[jax.ShapeDtypeStruct((1, 1), jnp.float32),
                 jax.ShapeDtypeStruct((1, 1), jnp.float32)],
  )(plog.reshape(128, 128), nlog.reshape(128, 128), regs.reshape(4, 128))
  return (cls[0, 0], reg[0, 0])
